# baseline (speedup 1.0000x reference)
import functools

import jax
import jax.numpy as jnp
from jax import lax
from jax.experimental import pallas as pl
from jax.experimental.pallas import tpu as pltpu
from jax.experimental.pallas import tpu_sc as plsc

NW = 32


def _embed_kernel(v, d, l, bt):
    lt = l // 8
    mesh = plsc.VectorSubcoreMesh(core_axis_name="c", subcore_axis_name="s")

    @functools.partial(
        pl.kernel,
        mesh=mesh,
        out_type=jax.ShapeDtypeStruct((l, d // 8, bt, 8, 128), jnp.float32),
        scratch_types=[
            pltpu.VMEM((lt, 8, 128), jnp.int32),
            pltpu.VMEM((2, 128, d), jnp.float32),
            pltpu.VMEM((2, d // 8, 8, 128), jnp.float32),
            [pltpu.SemaphoreType.DMA] * 2,
            [pltpu.SemaphoreType.DMA] * 2,
        ],
        compiler_params=pltpu.CompilerParams(
            use_tc_tiling_on_sc=False, needs_layout_passes=False),
    )
    def k(idx_hbm, table_hbm, out_hbm, idx_v, rows_v, t_v, gsems, wsems):
        wid = lax.axis_index("s") * 2 + lax.axis_index("c")
        pltpu.sync_copy(idx_hbm.at[:, wid], idx_v)

        def gather_args(u, b):
            return table_hbm.at[idx_v.at[u // 8, u % 8]], rows_v.at[b], gsems[b]

        def write_args(u, b):
            return t_v.at[b], out_hbm.at[u, :, wid], wsems[b]

        row_ids = [lax.iota(jnp.int32, 16) + 16 * i for i in range(8)]

        def slot(u, b):
            @pl.when(u >= 2)
            def _():
                pltpu.make_async_copy(*write_args(u - 2, b)).wait()

            @pl.when(u + 1 < l)
            def _():
                pltpu.async_copy(*gather_args(u + 1, 1 - b))

            pltpu.make_async_copy(*gather_args(u, b)).wait()
            for f in range(d):
                col_f = jnp.full((16,), f, jnp.int32)
                for i in range(8):
                    vals = plsc.load_gather(rows_v.at[b], [row_ids[i], col_f])
                    t_v[b, f // 8, f % 8, pl.ds(16 * i, 16)] = vals
            pltpu.async_copy(*write_args(u, b))

        pltpu.async_copy(*gather_args(0, 0))

        def body(g, carry):
            slot(2 * g, 0)
            slot(2 * g + 1, 1)
            return carry

        lax.fori_loop(0, l // 2, body, 0)

        for u in (l - 2, l - 1):
            pltpu.make_async_copy(*write_args(u, u % 2)).wait()

    return k


def kernel(indices, table):
    b, l = indices.shape
    v, d = table.shape
    lt, bt = l // 8, b // 128
    idx4 = jnp.transpose(indices).reshape(lt, 8, bt, 128).transpose(0, 2, 1, 3)
    out5 = _embed_kernel(v, d, l, bt)(idx4, table)
    return out5.transpose(2, 4, 0, 1, 3).reshape(b, l, d)


# speedup vs baseline: 1.8113x; 1.8113x over previous
import functools

import jax
import jax.numpy as jnp
from jax import lax
from jax.experimental import pallas as pl
from jax.experimental.pallas import tpu as pltpu
from jax.experimental.pallas import tpu_sc as plsc

NW = 32


def _embed_kernel(v, d, l, bt):
    lt = l // 8
    mesh = plsc.VectorSubcoreMesh(core_axis_name="c", subcore_axis_name="s")

    @functools.partial(
        pl.kernel,
        mesh=mesh,
        out_type=jax.ShapeDtypeStruct((l, d // 8, bt, 8, 128), jnp.float32),
        scratch_types=[
            pltpu.VMEM((lt, 8, 128), jnp.int32),
            pltpu.VMEM((2, 128, d), jnp.float32),
            pltpu.VMEM((2, d // 8, 8, 129), jnp.float32),
            [pltpu.SemaphoreType.DMA] * 2,
            [pltpu.SemaphoreType.DMA] * 2,
        ],
        compiler_params=pltpu.CompilerParams(
            use_tc_tiling_on_sc=False, needs_layout_passes=False),
    )
    def k(idx_hbm, table_hbm, out_hbm, idx_v, rows_v, t_v, gsems, wsems):
        wid = lax.axis_index("s") * 2 + lax.axis_index("c")
        pltpu.sync_copy(idx_hbm.at[:, wid], idx_v)

        def gather_args(u, b):
            return (table_hbm.at[idx_v.at[u // 8, u % 8]],
                    rows_v.at[b], gsems[b])

        def write_args(u, b):
            return (t_v.at[b].at[:, :, pl.ds(0, 128)],
                    out_hbm.at[u, :, wid], wsems[b])

        iota = lax.iota(jnp.int32, 16)
        f_ids = [(iota + f0) >> 3 for f0 in range(0, d, 16)]
        fs_ids = [(iota + f0) & 7 for f0 in range(0, d, 16)]

        def slot(u, b):
            @pl.when(u >= 2)
            def _():
                pltpu.make_async_copy(*write_args(u - 2, b)).wait()

            @pl.when(u + 1 < l)
            def _():
                pltpu.async_copy(*gather_args(u + 1, 1 - b))

            pltpu.make_async_copy(*gather_args(u, b)).wait()
            for r in range(128):
                col_r = jnp.full((16,), r, jnp.int32)
                for i in range(d // 16):
                    vals = rows_v[b, r, pl.ds(16 * i, 16)]
                    plsc.store_scatter(
                        t_v.at[b], [f_ids[i], fs_ids[i], col_r], vals)
            pltpu.async_copy(*write_args(u, b))

        pltpu.async_copy(*gather_args(0, 0))

        def body(g, carry):
            slot(2 * g, 0)
            slot(2 * g + 1, 1)
            return carry

        lax.fori_loop(0, l // 2, body, 0)

        for u in (l - 2, l - 1):
            pltpu.make_async_copy(*write_args(u, u % 2)).wait()

    return k


def kernel(indices, table):
    b, l = indices.shape
    v, d = table.shape
    lt, bt = l // 8, b // 128
    idx4 = jnp.transpose(indices).reshape(lt, 8, bt, 128).transpose(0, 2, 1, 3)
    out5 = _embed_kernel(v, d, l, bt)(idx4, table)
    return out5.transpose(2, 4, 0, 1, 3).reshape(b, l, d)


# speedup vs baseline: 1.8143x; 1.0017x over previous
import functools

import jax
import jax.numpy as jnp
from jax import lax
from jax.experimental import pallas as pl
from jax.experimental.pallas import tpu as pltpu
from jax.experimental.pallas import tpu_sc as plsc

NW = 32


def _embed_kernel(v, d, l, bt):
    lt = l // 8
    mesh = plsc.VectorSubcoreMesh(core_axis_name="c", subcore_axis_name="s")

    @functools.partial(
        pl.kernel,
        mesh=mesh,
        out_type=jax.ShapeDtypeStruct((l, d // 8, bt, 8, 128), jnp.float32),
        scratch_types=[
            pltpu.VMEM((lt, 8, 128), jnp.int32),
            pltpu.VMEM((2, 128, d), jnp.float32),
            pltpu.VMEM((2, d // 8, 8, 129), jnp.float32),
            [pltpu.SemaphoreType.DMA] * 2,
            [pltpu.SemaphoreType.DMA] * 2,
        ],
        compiler_params=pltpu.CompilerParams(
            use_tc_tiling_on_sc=False, needs_layout_passes=False),
    )
    def k(idx_hbm, table_hbm, out_hbm, idx_v, rows_v, t_v, gsems, wsems):
        wid = lax.axis_index("s") * 2 + lax.axis_index("c")
        pltpu.sync_copy(idx_hbm.at[:, wid], idx_v)

        def gather_args(u, b):
            return (table_hbm.at[idx_v.at[u // 8, u % 8]],
                    rows_v.at[b], gsems[b])

        def write_args(u, b):
            return (t_v.at[b].at[:, :, pl.ds(0, 128)],
                    out_hbm.at[u, :, wid], wsems[b])

        iota = lax.iota(jnp.int32, 16)
        f_ids = [(iota + f0) >> 3 for f0 in range(0, d, 16)]
        fs_ids = [(iota + f0) & 7 for f0 in range(0, d, 16)]

        def slot(u, b):
            @pl.when(u >= 2)
            def _():
                pltpu.make_async_copy(*write_args(u - 2, b)).wait()

            @pl.when(u + 1 < l)
            def _():
                pltpu.async_copy(*gather_args(u + 1, 1 - b))

            pltpu.make_async_copy(*gather_args(u, b)).wait()
            for r0 in range(0, 128, 4):
                vals = [rows_v[b, r0 + k, pl.ds(16 * i, 16)]
                        for k in range(4) for i in range(d // 16)]
                for k in range(4):
                    col_r = jnp.full((16,), r0 + k, jnp.int32)
                    for i in range(d // 16):
                        plsc.store_scatter(
                            t_v.at[b], [f_ids[i], fs_ids[i], col_r],
                            vals[k * (d // 16) + i])
            pltpu.async_copy(*write_args(u, b))

        pltpu.async_copy(*gather_args(0, 0))

        def body(g, carry):
            slot(2 * g, 0)
            slot(2 * g + 1, 1)
            return carry

        lax.fori_loop(0, l // 2, body, 0)

        for u in (l - 2, l - 1):
            pltpu.make_async_copy(*write_args(u, u % 2)).wait()

    return k


def kernel(indices, table):
    b, l = indices.shape
    v, d = table.shape
    lt, bt = l // 8, b // 128
    idx4 = jnp.transpose(indices).reshape(lt, 8, bt, 128).transpose(0, 2, 1, 3)
    out5 = _embed_kernel(v, d, l, bt)(idx4, table)
    return out5.transpose(2, 4, 0, 1, 3).reshape(b, l, d)
